# baseline (speedup 1.0000x reference)
import functools

import jax
import jax.numpy as jnp
from jax import lax
from jax.experimental import pallas as pl
from jax.experimental.pallas import tpu as pltpu
from jax.experimental.pallas import tpu_sc as plsc

NUM_WORKERS = 32
CB = 8
NBUF = 4


@functools.partial(jax.jit, static_argnums=(2, 3, 4))
def _gather_rows(indices, table, b, h, d):
    rows_per_w = b // NUM_WORKERS
    n_chunks = rows_per_w // CB
    n_groups = n_chunks // NBUF
    mesh = plsc.VectorSubcoreMesh(core_axis_name="c", subcore_axis_name="s")

    @functools.partial(
        pl.kernel,
        mesh=mesh,
        out_type=jax.ShapeDtypeStruct((b, h, d), jnp.float32),
        scratch_types=[
            pltpu.VMEM((rows_per_w, h), jnp.int32),
            [pltpu.VMEM((CB, h, d), jnp.float32) for _ in range(NBUF)],
            [pltpu.SemaphoreType.DMA for _ in range(NBUF)],
            [pltpu.SemaphoreType.DMA for _ in range(NBUF)],
        ],
        compiler_params=pltpu.CompilerParams(use_tc_tiling_on_sc=False),
    )
    def k(idx_hbm, table_hbm, out_hbm, idx_v, rows, semg, semw):
        wid = lax.axis_index("s") * 2 + lax.axis_index("c")
        base = wid * rows_per_w

        def start_gather(c, p):
            for k_ in range(CB):
                pltpu.async_copy(
                    table_hbm.at[idx_v.at[c * CB + k_]], rows[p].at[k_], semg[p]
                )

        def wait_gather(c, p):
            for k_ in range(CB):
                pltpu.make_async_copy(
                    table_hbm.at[idx_v.at[c * CB + k_]], rows[p].at[k_], semg[p]
                ).wait()

        def start_write(c, p):
            pltpu.async_copy(
                rows[p], out_hbm.at[pl.ds(base + c * CB, CB)], semw[p]
            )

        def wait_write(c, p):
            pltpu.make_async_copy(
                rows[p], out_hbm.at[pl.ds(base + c * CB, CB)], semw[p]
            ).wait()

        pltpu.sync_copy(idx_hbm.at[pl.ds(base, rows_per_w)], idx_v)

        for p in range(NBUF):
            start_gather(p, p)
            if p >= 2:
                wait_gather(p - 2, p - 2)
                start_write(p - 2, p - 2)

        def body(g, carry):
            for p in range(NBUF):
                c = g * NBUF + p
                wait_write(c - NBUF, p)
                start_gather(c, p)
                wait_gather(c - 2, (p - 2) % NBUF)
                start_write(c - 2, (p - 2) % NBUF)
            return carry

        lax.fori_loop(1, n_groups, body, 0)

        n = n_chunks
        for c in (n - 2, n - 1):
            p = c % NBUF
            wait_gather(c, p)
            start_write(c, p)
        for p in range(NBUF):
            wait_write(n - NBUF + p, p)

    return k(indices, table)


def kernel(indices, table):
    b, h = indices.shape
    v, d = table.shape
    return _gather_rows(indices, table, b, h, d)
